# Initial kernel scaffold; baseline (speedup 1.0000x reference)
import functools

import jax
import jax.numpy as jnp
from jax import lax
from jax.experimental import pallas as pl
from jax.experimental.pallas import tpu as pltpu
from jax.experimental.pallas import tpu_sc as plsc

NC = 2
NS = 16
NW = NC * NS
L = 16
B = 128
DEGW = 16
ZR = 64


def _mesh():
    return plsc.VectorSubcoreMesh(
        core_axis_name="c", subcore_axis_name="s",
        num_cores=NC, num_subcores=NS)


def _deg_body(npad, nchunks, dst_hbm, deg_out, dst_v, ones_v, zb, sdeg):
    c = lax.axis_index("c")
    s = lax.axis_index("s")
    w = c * NS + s
    rows = npad // NS

    def fill_zb(i, carry):
        zb[i, :] = jnp.zeros((L,), jnp.float32)
        return carry

    lax.fori_loop(0, rows, fill_zb, 0)

    def fill_ones(i, carry):
        ones_v[i, :] = jnp.ones((L,), jnp.float32)
        return carry

    lax.fori_loop(0, B, fill_ones, 0)

    pltpu.sync_copy(dst_hbm.at[w], dst_v)
    pltpu.sync_copy(zb, sdeg.at[pl.ds(s * rows, rows)])
    plsc.subcore_barrier()

    def scat(j, carry):
        pltpu.sync_copy(ones_v, sdeg.at[dst_v.at[j]], add=True)
        return carry

    lax.fori_loop(0, nchunks, scat, 0)
    plsc.subcore_barrier()
    pltpu.sync_copy(sdeg.at[pl.ds(s * rows, rows)],
                    deg_out.at[c, pl.ds(s * rows, rows)])


def _agg_body(npad, nchunks, d_in, xs_hbm, src_hbm, dst_hbm, acc_out,
              src_v, dst_v, buf0, buf1, zb, sacc, sem0, sem1):
    c = lax.axis_index("c")
    s = lax.axis_index("s")
    w = c * NS + s
    rows = npad // NS
    dsub = d_in // L

    def fill_zb(k, carry):
        zb[k // dsub, pl.ds((k % dsub) * L, L)] = jnp.zeros((L,), jnp.float32)
        return carry

    lax.fori_loop(0, ZR * dsub, fill_zb, 0)

    def zero_acc(k, carry):
        pltpu.sync_copy(zb, sacc.at[pl.ds(s * rows + k * ZR, ZR)])
        return carry

    lax.fori_loop(0, rows // ZR, zero_acc, 0)

    pltpu.sync_copy(src_hbm.at[w], src_v)
    pltpu.sync_copy(dst_hbm.at[w], dst_v)
    plsc.subcore_barrier()

    def pair(k, carry):
        j0 = 2 * k
        j1 = 2 * k + 1
        d0 = pltpu.async_copy(xs_hbm.at[src_v.at[j0]], buf0, sem0)
        d1 = pltpu.async_copy(xs_hbm.at[src_v.at[j1]], buf1, sem1)
        d0.wait()
        pltpu.sync_copy(buf0, sacc.at[dst_v.at[j0]], add=True)
        d1.wait()
        pltpu.sync_copy(buf1, sacc.at[dst_v.at[j1]], add=True)
        return carry

    lax.fori_loop(0, nchunks // 2, pair, 0)
    plsc.subcore_barrier()
    pltpu.sync_copy(sacc.at[pl.ds(s * rows, rows)],
                    acc_out.at[c, pl.ds(s * rows, rows)])


def _scale_body(deg_ref, x_ref, dinv_ref, xs_ref):
    d = deg_ref[:, 0:1] + deg_ref[:, DEGW:DEGW + 1] + 1.0
    dinv = lax.rsqrt(d)
    dinv_ref[...] = dinv
    xs_ref[...] = x_ref[...] * dinv


def _head_body(acc0_ref, acc1_ref, xs_ref, dinv_ref, wmu_ref, bmu_ref,
               wls_ref, bls_ref, mu_ref, ls_ref):
    s = (acc0_ref[...] + acc1_ref[...] + xs_ref[...]) * dinv_ref[...]
    mu_ref[...] = (
        jnp.dot(s, wmu_ref[...], preferred_element_type=jnp.float32)
        + bmu_ref[...])
    ls_ref[...] = (
        jnp.dot(s, wls_ref[...], preferred_element_type=jnp.float32)
        + bls_ref[...])


def kernel(x, edge_index, W_mu, b_mu, W_logstd, b_logstd):
    N, d_in = x.shape
    d_out = W_mu.shape[1]
    E = edge_index.shape[1]

    npad = ((N + 1 + NS * ZR - 1) // (NS * ZR)) * (NS * ZR)
    nchunks = -(-E // (NW * B))
    nchunks += nchunks % 2
    epad = NW * B * nchunks

    src = edge_index[0].astype(jnp.int32)
    dst = edge_index[1].astype(jnp.int32)
    fill = jnp.full((epad - E,), N, jnp.int32)
    src_r = jnp.concatenate([src, fill]).reshape(NW, nchunks, B)
    dst_r = jnp.concatenate([dst, fill]).reshape(NW, nchunks, B)
    x_pad = jnp.pad(x, ((0, npad - N), (0, 0)))

    deg = pl.kernel(
        functools.partial(_deg_body, npad, nchunks),
        out_type=jax.ShapeDtypeStruct((NC, npad, DEGW), jnp.float32),
        mesh=_mesh(),
        scratch_types=[
            pltpu.VMEM((nchunks, B), jnp.int32),
            pltpu.VMEM((B, DEGW), jnp.float32),
            pltpu.VMEM((npad // NS, DEGW), jnp.float32),
            pltpu.VMEM_SHARED((npad, DEGW), jnp.float32),
        ],
        name="gcn_degree_sc",
    )(dst_r)

    deg_c = deg.transpose(1, 0, 2).reshape(npad, NC * DEGW)

    RB = 1024
    dinv, xs = pl.pallas_call(
        _scale_body,
        grid=(npad // RB,),
        in_specs=[
            pl.BlockSpec((RB, NC * DEGW), lambda i: (i, 0)),
            pl.BlockSpec((RB, d_in), lambda i: (i, 0)),
        ],
        out_specs=[
            pl.BlockSpec((RB, 1), lambda i: (i, 0)),
            pl.BlockSpec((RB, d_in), lambda i: (i, 0)),
        ],
        out_shape=[
            jax.ShapeDtypeStruct((npad, 1), jnp.float32),
            jax.ShapeDtypeStruct((npad, d_in), jnp.float32),
        ],
        name="gcn_scale_tc",
    )(deg_c, x_pad)

    acc = pl.kernel(
        functools.partial(_agg_body, npad, nchunks, d_in),
        out_type=jax.ShapeDtypeStruct((NC, npad, d_in), jnp.float32),
        mesh=_mesh(),
        scratch_types=[
            pltpu.VMEM((nchunks, B), jnp.int32),
            pltpu.VMEM((nchunks, B), jnp.int32),
            pltpu.VMEM((B, d_in), jnp.float32),
            pltpu.VMEM((B, d_in), jnp.float32),
            pltpu.VMEM((ZR, d_in), jnp.float32),
            pltpu.VMEM_SHARED((npad, d_in), jnp.float32),
            pltpu.SemaphoreType.DMA,
            pltpu.SemaphoreType.DMA,
        ],
        name="gcn_aggregate_sc",
    )(xs, src_r, dst_r)

    RO = 2000 if N % 2000 == 0 else (1000 if N % 1000 == 0 else N)
    mu, ls = pl.pallas_call(
        _head_body,
        grid=(N // RO,),
        in_specs=[
            pl.BlockSpec((None, RO, d_in), lambda i: (0, i, 0)),
            pl.BlockSpec((None, RO, d_in), lambda i: (1, i, 0)),
            pl.BlockSpec((RO, d_in), lambda i: (i, 0)),
            pl.BlockSpec((RO, 1), lambda i: (i, 0)),
            pl.BlockSpec((d_in, d_out), lambda i: (0, 0)),
            pl.BlockSpec((1, d_out), lambda i: (0, 0)),
            pl.BlockSpec((d_in, d_out), lambda i: (0, 0)),
            pl.BlockSpec((1, d_out), lambda i: (0, 0)),
        ],
        out_specs=[
            pl.BlockSpec((RO, d_out), lambda i: (i, 0)),
            pl.BlockSpec((RO, d_out), lambda i: (i, 0)),
        ],
        out_shape=[
            jax.ShapeDtypeStruct((N, d_out), jnp.float32),
            jax.ShapeDtypeStruct((N, d_out), jnp.float32),
        ],
        name="gcn_heads_tc",
    )(acc, acc, xs, dinv, W_mu, b_mu.reshape(1, -1),
      W_logstd, b_logstd.reshape(1, -1))
    return (mu, ls)


# speedup vs baseline: 14.6538x; 14.6538x over previous
import functools

import jax
import jax.numpy as jnp
from jax import lax
from jax.experimental import pallas as pl
from jax.experimental.pallas import tpu as pltpu
from jax.experimental.pallas import tpu_sc as plsc

NC = 2
NS = 16
NW = NC * NS
L = 16
B = 128
ZR = 16


def _mesh():
    return plsc.VectorSubcoreMesh(
        core_axis_name="c", subcore_axis_name="s",
        num_cores=NC, num_subcores=NS)


def _fill2d(ref, rows, width, value):
    sub = width // L

    def body(k, carry):
        ref[k // sub, pl.ds(lax.rem(k, sub) * L, L)] = jnp.full(
            (L,), value, jnp.float32)
        return carry

    lax.fori_loop(0, rows * sub, body, 0)


def _deg_body(npad, nchunks, d_in, dst_hbm, deg_out, idx_v, ones_v, zb, sdeg):
    c = lax.axis_index("c")
    s = lax.axis_index("s")
    w = c * NS + s
    rows = npad // NS

    _fill2d(zb, ZR, d_in, 0.0)
    _fill2d(ones_v, B, d_in, 1.0)

    def zero_acc(k, carry):
        pltpu.sync_copy(zb, sdeg.at[pl.ds(s * rows + k * ZR, ZR)])
        return carry

    lax.fori_loop(0, rows // ZR, zero_acc, 0)
    plsc.subcore_barrier()

    def scat(j, carry):
        pltpu.sync_copy(dst_hbm.at[w, j], idx_v)
        pltpu.sync_copy(ones_v, sdeg.at[idx_v], add=True)
        return carry

    lax.fori_loop(0, nchunks, scat, 0)
    plsc.subcore_barrier()
    pltpu.sync_copy(sdeg.at[pl.ds(s * rows, rows)],
                    deg_out.at[c, pl.ds(s * rows, rows)])


def _agg_body(npad, nchunks, d_in, xs_hbm, src_hbm, dst_hbm, acc_out,
              sidx0, sidx1, didx0, didx1, buf0, buf1, zb, sacc, sem0, sem1):
    c = lax.axis_index("c")
    s = lax.axis_index("s")
    w = c * NS + s
    rows = npad // NS

    _fill2d(zb, ZR, d_in, 0.0)

    def zero_acc(k, carry):
        pltpu.sync_copy(zb, sacc.at[pl.ds(s * rows + k * ZR, ZR)])
        return carry

    lax.fori_loop(0, rows // ZR, zero_acc, 0)
    plsc.subcore_barrier()

    def pair(k, carry):
        j0 = 2 * k
        j1 = 2 * k + 1
        pltpu.sync_copy(src_hbm.at[w, j0], sidx0)
        d0 = pltpu.async_copy(xs_hbm.at[sidx0], buf0, sem0)
        pltpu.sync_copy(src_hbm.at[w, j1], sidx1)
        d1 = pltpu.async_copy(xs_hbm.at[sidx1], buf1, sem1)
        pltpu.sync_copy(dst_hbm.at[w, j0], didx0)
        d0.wait()
        pltpu.sync_copy(buf0, sacc.at[didx0], add=True)
        pltpu.sync_copy(dst_hbm.at[w, j1], didx1)
        d1.wait()
        pltpu.sync_copy(buf1, sacc.at[didx1], add=True)
        return carry

    lax.fori_loop(0, nchunks // 2, pair, 0)
    plsc.subcore_barrier()
    pltpu.sync_copy(sacc.at[pl.ds(s * rows, rows)],
                    acc_out.at[c, pl.ds(s * rows, rows)])


def _scale_body(deg0_ref, deg1_ref, x_ref, dinv_ref, xs_ref):
    d = deg0_ref[:, 0:1] + deg1_ref[:, 0:1] + 1.0
    dinv = lax.rsqrt(d)
    dinv_ref[...] = dinv
    xs_ref[...] = x_ref[...] * dinv


def _head_body(acc0_ref, acc1_ref, xs_ref, dinv_ref, wmu_ref, bmu_ref,
               wls_ref, bls_ref, mu_ref, ls_ref):
    s = (acc0_ref[...] + acc1_ref[...] + xs_ref[...]) * dinv_ref[...]
    mu_ref[...] = (
        jnp.dot(s, wmu_ref[...], preferred_element_type=jnp.float32)
        + bmu_ref[...])
    ls_ref[...] = (
        jnp.dot(s, wls_ref[...], preferred_element_type=jnp.float32)
        + bls_ref[...])


def kernel(x, edge_index, W_mu, b_mu, W_logstd, b_logstd):
    N, d_in = x.shape
    d_out = W_mu.shape[1]
    E = edge_index.shape[1]

    npad = ((N + 1 + NS * ZR - 1) // (NS * ZR)) * (NS * ZR)
    nchunks = -(-E // (NW * B))
    nchunks += nchunks % 2
    epad = NW * B * nchunks

    src = edge_index[0].astype(jnp.int32)
    dst = edge_index[1].astype(jnp.int32)
    fill = jnp.full((epad - E,), N, jnp.int32)
    src_r = jnp.concatenate([src, fill]).reshape(NW, nchunks, B)
    dst_r = jnp.concatenate([dst, fill]).reshape(NW, nchunks, B)
    x_pad = jnp.pad(x, ((0, npad - N), (0, 0)))

    deg = pl.kernel(
        functools.partial(_deg_body, npad, nchunks, d_in),
        out_type=jax.ShapeDtypeStruct((NC, npad, d_in), jnp.float32),
        mesh=_mesh(),
        scratch_types=[
            pltpu.VMEM((B,), jnp.int32),
            pltpu.VMEM((B, d_in), jnp.float32),
            pltpu.VMEM((ZR, d_in), jnp.float32),
            pltpu.VMEM_SHARED((npad, d_in), jnp.float32),
        ],
        name="gcn_degree_sc",
    )(dst_r)

    RB = 1024
    dinv, xs = pl.pallas_call(
        _scale_body,
        grid=(npad // RB,),
        in_specs=[
            pl.BlockSpec((None, RB, d_in), lambda i: (0, i, 0)),
            pl.BlockSpec((None, RB, d_in), lambda i: (1, i, 0)),
            pl.BlockSpec((RB, d_in), lambda i: (i, 0)),
        ],
        out_specs=[
            pl.BlockSpec((RB, 1), lambda i: (i, 0)),
            pl.BlockSpec((RB, d_in), lambda i: (i, 0)),
        ],
        out_shape=[
            jax.ShapeDtypeStruct((npad, 1), jnp.float32),
            jax.ShapeDtypeStruct((npad, d_in), jnp.float32),
        ],
        name="gcn_scale_tc",
    )(deg, deg, x_pad)

    acc = pl.kernel(
        functools.partial(_agg_body, npad, nchunks, d_in),
        out_type=jax.ShapeDtypeStruct((NC, npad, d_in), jnp.float32),
        mesh=_mesh(),
        scratch_types=[
            pltpu.VMEM((B,), jnp.int32),
            pltpu.VMEM((B,), jnp.int32),
            pltpu.VMEM((B,), jnp.int32),
            pltpu.VMEM((B,), jnp.int32),
            pltpu.VMEM((B, d_in), jnp.float32),
            pltpu.VMEM((B, d_in), jnp.float32),
            pltpu.VMEM((ZR, d_in), jnp.float32),
            pltpu.VMEM_SHARED((npad, d_in), jnp.float32),
            pltpu.SemaphoreType.DMA,
            pltpu.SemaphoreType.DMA,
        ],
        name="gcn_aggregate_sc",
    )(xs, src_r, dst_r)

    RO = 2000 if N % 2000 == 0 else (1000 if N % 1000 == 0 else N)
    mu, ls = pl.pallas_call(
        _head_body,
        grid=(N // RO,),
        in_specs=[
            pl.BlockSpec((None, RO, d_in), lambda i: (0, i, 0)),
            pl.BlockSpec((None, RO, d_in), lambda i: (1, i, 0)),
            pl.BlockSpec((RO, d_in), lambda i: (i, 0)),
            pl.BlockSpec((RO, 1), lambda i: (i, 0)),
            pl.BlockSpec((d_in, d_out), lambda i: (0, 0)),
            pl.BlockSpec((1, d_out), lambda i: (0, 0)),
            pl.BlockSpec((d_in, d_out), lambda i: (0, 0)),
            pl.BlockSpec((1, d_out), lambda i: (0, 0)),
        ],
        out_specs=[
            pl.BlockSpec((RO, d_out), lambda i: (i, 0)),
            pl.BlockSpec((RO, d_out), lambda i: (i, 0)),
        ],
        out_shape=[
            jax.ShapeDtypeStruct((N, d_out), jnp.float32),
            jax.ShapeDtypeStruct((N, d_out), jnp.float32),
        ],
        name="gcn_heads_tc",
    )(acc, acc, xs, dinv, W_mu, b_mu.reshape(1, -1),
      W_logstd, b_logstd.reshape(1, -1))
    return (mu, ls)


# speedup vs baseline: 14.9113x; 1.0176x over previous
import functools

import jax
import jax.numpy as jnp
from jax import lax
from jax.experimental import pallas as pl
from jax.experimental.pallas import tpu as pltpu
from jax.experimental.pallas import tpu_sc as plsc

NC = 2
NS = 16
NW = NC * NS
L = 16
B = 128
ZR = 16
GS = 8


def _mesh():
    return plsc.VectorSubcoreMesh(
        core_axis_name="c", subcore_axis_name="s",
        num_cores=NC, num_subcores=NS)


def _fill2d(ref, rows, width, value):
    sub = width // L

    def body(k, carry):
        ref[k // sub, pl.ds(lax.rem(k, sub) * L, L)] = jnp.full(
            (L,), value, jnp.float32)
        return carry

    lax.fori_loop(0, rows * sub, body, 0)


def _deg_body(npad, nchunks, d_in, dst_hbm, deg_out, idx_v, ones_v, zb, sdeg):
    c = lax.axis_index("c")
    s = lax.axis_index("s")
    w = c * NS + s
    rows = npad // NS

    _fill2d(zb, ZR, d_in, 0.0)
    _fill2d(ones_v, B, d_in, 1.0)

    def zero_acc(k, carry):
        pltpu.sync_copy(zb, sdeg.at[pl.ds(s * rows + k * ZR, ZR)])
        return carry

    lax.fori_loop(0, rows // ZR, zero_acc, 0)
    plsc.subcore_barrier()

    def scat(j, carry):
        pltpu.sync_copy(dst_hbm.at[w, j], idx_v)
        pltpu.sync_copy(ones_v, sdeg.at[idx_v], add=True)
        return carry

    lax.fori_loop(0, nchunks, scat, 0)
    plsc.subcore_barrier()
    pltpu.sync_copy(sdeg.at[pl.ds(s * rows, rows)],
                    deg_out.at[c, pl.ds(s * rows, rows)])


def _agg_body(npad, nchunks, d_in, xs_hbm, src_hbm, dst_hbm, acc_out,
              srcg0, srcg1, dstg0, dstg1, buf0, buf1, zb, sacc,
              semg0, semg1, sems0, sems1):
    c = lax.axis_index("c")
    s = lax.axis_index("s")
    w = c * NS + s
    rows = npad // NS

    _fill2d(zb, ZR, d_in, 0.0)

    def zero_acc(k, carry):
        pltpu.sync_copy(zb, sacc.at[pl.ds(s * rows + k * ZR, ZR)])
        return carry

    lax.fori_loop(0, rows // ZR, zero_acc, 0)
    plsc.subcore_barrier()

    srcg = (srcg0, srcg1)
    dstg = (dstg0, dstg1)
    sems = (sems0, sems1)
    semg = (semg0, semg1)
    bufs = (buf0, buf1)

    def gpair(k, carry):
        for parity in (0, 1):
            g = 2 * k + parity
            sg = srcg[parity]
            dg = dstg[parity]
            pltpu.sync_copy(src_hbm.at[w, pl.ds(g * GS, GS)], sg)
            pltpu.sync_copy(dst_hbm.at[w, pl.ds(g * GS, GS)], dg)
            for i in range(GS // 2):
                gds = []
                for half in (0, 1):
                    j = 2 * i + half

                    def drain(_j=j, _half=half, _dg=dg):
                        pltpu.make_async_copy(
                            bufs[_half], sacc.at[_dg.at[_j]], sems[_half]
                        ).wait()

                    if parity == 0 and i == 0 and half == 0:
                        pl.when(k > 0)(drain)
                    elif parity == 0 and i == 0 and half == 1:
                        pl.when(k > 0)(drain)
                    else:
                        drain()
                    gds.append(pltpu.async_copy(
                        xs_hbm.at[sg.at[j]], bufs[half], semg[half]))
                for half in (0, 1):
                    j = 2 * i + half
                    gds[half].wait()
                    pltpu.async_copy(
                        bufs[half], sacc.at[dg.at[j]], sems[half], add=True)
        return carry

    lax.fori_loop(0, nchunks // (2 * GS), gpair, 0)
    for half in (0, 1):
        pltpu.make_async_copy(
            bufs[half], sacc.at[dstg1.at[GS - 2 + half]], sems[half]).wait()
    plsc.subcore_barrier()
    pltpu.sync_copy(sacc.at[pl.ds(s * rows, rows)],
                    acc_out.at[c, pl.ds(s * rows, rows)])


def _scale_body(deg0_ref, deg1_ref, x_ref, dinv_ref, xs_ref):
    d = deg0_ref[:, 0:1] + deg1_ref[:, 0:1] + 1.0
    dinv = lax.rsqrt(d)
    dinv_ref[...] = dinv
    xs_ref[...] = x_ref[...] * dinv


def _head_body(acc0_ref, acc1_ref, xs_ref, dinv_ref, wmu_ref, bmu_ref,
               wls_ref, bls_ref, mu_ref, ls_ref):
    s = (acc0_ref[...] + acc1_ref[...] + xs_ref[...]) * dinv_ref[...]
    mu_ref[...] = (
        jnp.dot(s, wmu_ref[...], preferred_element_type=jnp.float32)
        + bmu_ref[...])
    ls_ref[...] = (
        jnp.dot(s, wls_ref[...], preferred_element_type=jnp.float32)
        + bls_ref[...])


def kernel(x, edge_index, W_mu, b_mu, W_logstd, b_logstd):
    N, d_in = x.shape
    d_out = W_mu.shape[1]
    E = edge_index.shape[1]

    npad = ((N + 1 + NS * ZR - 1) // (NS * ZR)) * (NS * ZR)
    nchunks = -(-E // (NW * B))
    nchunks = ((nchunks + 2 * GS - 1) // (2 * GS)) * (2 * GS)
    epad = NW * B * nchunks

    src = edge_index[0].astype(jnp.int32)
    dst = edge_index[1].astype(jnp.int32)
    fill = jnp.full((epad - E,), N, jnp.int32)
    src_r = jnp.concatenate([src, fill]).reshape(NW, nchunks, B)
    dst_r = jnp.concatenate([dst, fill]).reshape(NW, nchunks, B)
    x_pad = jnp.pad(x, ((0, npad - N), (0, 0)))

    deg = pl.kernel(
        functools.partial(_deg_body, npad, nchunks, d_in),
        out_type=jax.ShapeDtypeStruct((NC, npad, d_in), jnp.float32),
        mesh=_mesh(),
        scratch_types=[
            pltpu.VMEM((B,), jnp.int32),
            pltpu.VMEM((B, d_in), jnp.float32),
            pltpu.VMEM((ZR, d_in), jnp.float32),
            pltpu.VMEM_SHARED((npad, d_in), jnp.float32),
        ],
        name="gcn_degree_sc",
    )(dst_r)

    RB = 1024
    dinv, xs = pl.pallas_call(
        _scale_body,
        grid=(npad // RB,),
        in_specs=[
            pl.BlockSpec((None, RB, d_in), lambda i: (0, i, 0)),
            pl.BlockSpec((None, RB, d_in), lambda i: (1, i, 0)),
            pl.BlockSpec((RB, d_in), lambda i: (i, 0)),
        ],
        out_specs=[
            pl.BlockSpec((RB, 1), lambda i: (i, 0)),
            pl.BlockSpec((RB, d_in), lambda i: (i, 0)),
        ],
        out_shape=[
            jax.ShapeDtypeStruct((npad, 1), jnp.float32),
            jax.ShapeDtypeStruct((npad, d_in), jnp.float32),
        ],
        name="gcn_scale_tc",
    )(deg, deg, x_pad)

    acc = pl.kernel(
        functools.partial(_agg_body, npad, nchunks, d_in),
        out_type=jax.ShapeDtypeStruct((NC, npad, d_in), jnp.float32),
        mesh=_mesh(),
        scratch_types=[
            pltpu.VMEM((GS, B), jnp.int32),
            pltpu.VMEM((GS, B), jnp.int32),
            pltpu.VMEM((GS, B), jnp.int32),
            pltpu.VMEM((GS, B), jnp.int32),
            pltpu.VMEM((B, d_in), jnp.float32),
            pltpu.VMEM((B, d_in), jnp.float32),
            pltpu.VMEM((ZR, d_in), jnp.float32),
            pltpu.VMEM_SHARED((npad, d_in), jnp.float32),
            pltpu.SemaphoreType.DMA,
            pltpu.SemaphoreType.DMA,
            pltpu.SemaphoreType.DMA,
            pltpu.SemaphoreType.DMA,
        ],
        name="gcn_aggregate_sc",
    )(xs, src_r, dst_r)

    RO = 2000 if N % 2000 == 0 else (1000 if N % 1000 == 0 else N)
    mu, ls = pl.pallas_call(
        _head_body,
        grid=(N // RO,),
        in_specs=[
            pl.BlockSpec((None, RO, d_in), lambda i: (0, i, 0)),
            pl.BlockSpec((None, RO, d_in), lambda i: (1, i, 0)),
            pl.BlockSpec((RO, d_in), lambda i: (i, 0)),
            pl.BlockSpec((RO, 1), lambda i: (i, 0)),
            pl.BlockSpec((d_in, d_out), lambda i: (0, 0)),
            pl.BlockSpec((1, d_out), lambda i: (0, 0)),
            pl.BlockSpec((d_in, d_out), lambda i: (0, 0)),
            pl.BlockSpec((1, d_out), lambda i: (0, 0)),
        ],
        out_specs=[
            pl.BlockSpec((RO, d_out), lambda i: (i, 0)),
            pl.BlockSpec((RO, d_out), lambda i: (i, 0)),
        ],
        out_shape=[
            jax.ShapeDtypeStruct((N, d_out), jnp.float32),
            jax.ShapeDtypeStruct((N, d_out), jnp.float32),
        ],
        name="gcn_heads_tc",
    )(acc, acc, xs, dinv, W_mu, b_mu.reshape(1, -1),
      W_logstd, b_logstd.reshape(1, -1))
    return (mu, ls)


# speedup vs baseline: 14.9394x; 1.0019x over previous
import functools

import jax
import jax.numpy as jnp
from jax import lax
from jax.experimental import pallas as pl
from jax.experimental.pallas import tpu as pltpu
from jax.experimental.pallas import tpu_sc as plsc

NC = 2
NS = 16
NW = NC * NS
L = 16
B = 128
ZR = 16
GS = 8


def _mesh():
    return plsc.VectorSubcoreMesh(
        core_axis_name="c", subcore_axis_name="s",
        num_cores=NC, num_subcores=NS)


def _fill2d(ref, rows, width, value):
    sub = width // L

    def body(k, carry):
        ref[k // sub, pl.ds(lax.rem(k, sub) * L, L)] = jnp.full(
            (L,), value, jnp.float32)
        return carry

    lax.fori_loop(0, rows * sub, body, 0)


def _deg_body(npad, nchunks, d_in, dst_hbm, deg_out, idx_v, ones_v, zb, sdeg):
    c = lax.axis_index("c")
    s = lax.axis_index("s")
    w = c * NS + s
    rows = npad // NS

    _fill2d(zb, ZR, d_in, 0.0)
    _fill2d(ones_v, B, d_in, 1.0)

    def zero_acc(k, carry):
        pltpu.sync_copy(zb, sdeg.at[pl.ds(s * rows + k * ZR, ZR)])
        return carry

    lax.fori_loop(0, rows // ZR, zero_acc, 0)
    plsc.subcore_barrier()

    def scat(j, carry):
        pltpu.sync_copy(dst_hbm.at[w, j], idx_v)
        pltpu.sync_copy(ones_v, sdeg.at[idx_v], add=True)
        return carry

    lax.fori_loop(0, nchunks, scat, 0)
    plsc.subcore_barrier()
    pltpu.sync_copy(sdeg.at[pl.ds(s * rows, rows)],
                    deg_out.at[c, pl.ds(s * rows, rows)])


def _agg_body(npad, nchunks, d_in, xs_hbm, src_hbm, dst_hbm, acc_out,
              srcg0, srcg1, dstg0, dstg1, buf0, buf1, zb, sacc,
              semg0, semg1, sems0, sems1, semh0, semh1):
    c = lax.axis_index("c")
    s = lax.axis_index("s")
    w = c * NS + s
    rows = npad // NS

    _fill2d(zb, ZR, d_in, 0.0)

    def zero_acc(k, carry):
        pltpu.sync_copy(zb, sacc.at[pl.ds(s * rows + k * ZR, ZR)])
        return carry

    lax.fori_loop(0, rows // ZR, zero_acc, 0)
    plsc.subcore_barrier()

    srcg = (srcg0, srcg1)
    dstg = (dstg0, dstg1)
    sems = (sems0, sems1)
    semg = (semg0, semg1)
    semh = (semh0, semh1)
    bufs = (buf0, buf1)
    H = B // 2

    def gpair(k, carry):
        for parity in (0, 1):
            g = 2 * k + parity
            sg = srcg[parity]
            dg = dstg[parity]
            pltpu.sync_copy(src_hbm.at[w, pl.ds(g * GS, GS)], sg)
            pltpu.sync_copy(dst_hbm.at[w, pl.ds(g * GS, GS)], dg)
            for i in range(GS // 2):
                gds = []
                for half in (0, 1):
                    j = 2 * i + half

                    def drain(_j=j, _half=half, _dg=dg):
                        pltpu.make_async_copy(
                            bufs[_half], sacc.at[_dg.at[_j]], sems[_half]
                        ).wait()

                    if parity == 0 and i == 0 and half == 0:
                        pl.when(k > 0)(drain)
                    elif parity == 0 and i == 0 and half == 1:
                        pl.when(k > 0)(drain)
                    else:
                        drain()
                    gds.append((
                        pltpu.async_copy(
                            xs_hbm.at[sg.at[j, pl.ds(0, H)]],
                            bufs[half].at[pl.ds(0, H)], semg[half]),
                        pltpu.async_copy(
                            xs_hbm.at[sg.at[j, pl.ds(H, H)]],
                            bufs[half].at[pl.ds(H, H)], semh[half]),
                    ))
                for half in (0, 1):
                    j = 2 * i + half
                    gds[half][0].wait()
                    gds[half][1].wait()
                    pltpu.async_copy(
                        bufs[half], sacc.at[dg.at[j]], sems[half], add=True)
        return carry

    lax.fori_loop(0, nchunks // (2 * GS), gpair, 0)
    for half in (0, 1):
        pltpu.make_async_copy(
            bufs[half], sacc.at[dstg1.at[GS - 2 + half]], sems[half]).wait()
    plsc.subcore_barrier()
    pltpu.sync_copy(sacc.at[pl.ds(s * rows, rows)],
                    acc_out.at[c, pl.ds(s * rows, rows)])


def _scale_body(deg0_ref, deg1_ref, x_ref, dinv_ref, xs_ref):
    d = deg0_ref[:, 0:1] + deg1_ref[:, 0:1] + 1.0
    dinv = lax.rsqrt(d)
    dinv_ref[...] = dinv
    xs_ref[...] = x_ref[...] * dinv


def _head_body(acc0_ref, acc1_ref, xs_ref, dinv_ref, wmu_ref, bmu_ref,
               wls_ref, bls_ref, mu_ref, ls_ref):
    s = (acc0_ref[...] + acc1_ref[...] + xs_ref[...]) * dinv_ref[...]
    mu_ref[...] = (
        jnp.dot(s, wmu_ref[...], preferred_element_type=jnp.float32)
        + bmu_ref[...])
    ls_ref[...] = (
        jnp.dot(s, wls_ref[...], preferred_element_type=jnp.float32)
        + bls_ref[...])


def kernel(x, edge_index, W_mu, b_mu, W_logstd, b_logstd):
    N, d_in = x.shape
    d_out = W_mu.shape[1]
    E = edge_index.shape[1]

    npad = ((N + 1 + NS * ZR - 1) // (NS * ZR)) * (NS * ZR)
    nchunks = -(-E // (NW * B))
    nchunks = ((nchunks + 2 * GS - 1) // (2 * GS)) * (2 * GS)
    epad = NW * B * nchunks

    src = edge_index[0].astype(jnp.int32)
    dst = edge_index[1].astype(jnp.int32)
    fill = jnp.full((epad - E,), N, jnp.int32)
    src_r = jnp.concatenate([src, fill]).reshape(NW, nchunks, B)
    dst_r = jnp.concatenate([dst, fill]).reshape(NW, nchunks, B)
    x_pad = jnp.pad(x, ((0, npad - N), (0, 0)))

    deg = pl.kernel(
        functools.partial(_deg_body, npad, nchunks, d_in),
        out_type=jax.ShapeDtypeStruct((NC, npad, d_in), jnp.float32),
        mesh=_mesh(),
        scratch_types=[
            pltpu.VMEM((B,), jnp.int32),
            pltpu.VMEM((B, d_in), jnp.float32),
            pltpu.VMEM((ZR, d_in), jnp.float32),
            pltpu.VMEM_SHARED((npad, d_in), jnp.float32),
        ],
        name="gcn_degree_sc",
    )(dst_r)

    RB = 1024
    dinv, xs = pl.pallas_call(
        _scale_body,
        grid=(npad // RB,),
        in_specs=[
            pl.BlockSpec((None, RB, d_in), lambda i: (0, i, 0)),
            pl.BlockSpec((None, RB, d_in), lambda i: (1, i, 0)),
            pl.BlockSpec((RB, d_in), lambda i: (i, 0)),
        ],
        out_specs=[
            pl.BlockSpec((RB, 1), lambda i: (i, 0)),
            pl.BlockSpec((RB, d_in), lambda i: (i, 0)),
        ],
        out_shape=[
            jax.ShapeDtypeStruct((npad, 1), jnp.float32),
            jax.ShapeDtypeStruct((npad, d_in), jnp.float32),
        ],
        name="gcn_scale_tc",
    )(deg, deg, x_pad)

    acc = pl.kernel(
        functools.partial(_agg_body, npad, nchunks, d_in),
        out_type=jax.ShapeDtypeStruct((NC, npad, d_in), jnp.float32),
        mesh=_mesh(),
        scratch_types=[
            pltpu.VMEM((GS, B), jnp.int32),
            pltpu.VMEM((GS, B), jnp.int32),
            pltpu.VMEM((GS, B), jnp.int32),
            pltpu.VMEM((GS, B), jnp.int32),
            pltpu.VMEM((B, d_in), jnp.float32),
            pltpu.VMEM((B, d_in), jnp.float32),
            pltpu.VMEM((ZR, d_in), jnp.float32),
            pltpu.VMEM_SHARED((npad, d_in), jnp.float32),
            pltpu.SemaphoreType.DMA,
            pltpu.SemaphoreType.DMA,
            pltpu.SemaphoreType.DMA,
            pltpu.SemaphoreType.DMA,
            pltpu.SemaphoreType.DMA,
            pltpu.SemaphoreType.DMA,
        ],
        name="gcn_aggregate_sc",
    )(xs, src_r, dst_r)

    RO = 2000 if N % 2000 == 0 else (1000 if N % 1000 == 0 else N)
    mu, ls = pl.pallas_call(
        _head_body,
        grid=(N // RO,),
        in_specs=[
            pl.BlockSpec((None, RO, d_in), lambda i: (0, i, 0)),
            pl.BlockSpec((None, RO, d_in), lambda i: (1, i, 0)),
            pl.BlockSpec((RO, d_in), lambda i: (i, 0)),
            pl.BlockSpec((RO, 1), lambda i: (i, 0)),
            pl.BlockSpec((d_in, d_out), lambda i: (0, 0)),
            pl.BlockSpec((1, d_out), lambda i: (0, 0)),
            pl.BlockSpec((d_in, d_out), lambda i: (0, 0)),
            pl.BlockSpec((1, d_out), lambda i: (0, 0)),
        ],
        out_specs=[
            pl.BlockSpec((RO, d_out), lambda i: (i, 0)),
            pl.BlockSpec((RO, d_out), lambda i: (i, 0)),
        ],
        out_shape=[
            jax.ShapeDtypeStruct((N, d_out), jnp.float32),
            jax.ShapeDtypeStruct((N, d_out), jnp.float32),
        ],
        name="gcn_heads_tc",
    )(acc, acc, xs, dinv, W_mu, b_mu.reshape(1, -1),
      W_logstd, b_logstd.reshape(1, -1))
    return (mu, ls)
